# baseline (speedup 1.0000x reference)
import functools

import jax
import jax.numpy as jnp
from jax import lax
from jax.experimental import pallas as pl
from jax.experimental.pallas import tpu as pltpu
from jax.experimental.pallas import tpu_sc as plsc

NUM_STUDENTS = 1000000
NUM_QUESTIONS = 100000
NUM_DIM = 64
BATCH = 16384

NC = 2
NS = 16
L = 16
NW = NC * NS
B_PER_W = BATCH // NW
CHUNK = 128
N_CHUNKS = B_PER_W // CHUNK
GROUPS = B_PER_W // L


def _body(sid_hbm, qid_hbm, theta_hbm, beta_hbm, out_hbm,
          sid_v, qid_v, theta_v, beta_v, tpad_v, out_v, sem):
    wid = lax.axis_index("s") * NC + lax.axis_index("c")

    pltpu.sync_copy(sid_hbm.at[pl.ds(wid * N_CHUNKS, N_CHUNKS)], sid_v)
    pltpu.sync_copy(qid_hbm.at[pl.ds(wid * N_CHUNKS, N_CHUNKS)], qid_v)

    copies = []
    for j in range(N_CHUNKS):
        copies.append(pltpu.async_copy(
            theta_hbm.at[sid_v.at[j]],
            theta_v.at[pl.ds(j * CHUNK, CHUNK)], sem))
        copies.append(pltpu.async_copy(
            beta_hbm.at[qid_v.at[j]],
            beta_v.at[pl.ds(j * CHUNK, CHUNK)], sem))
    for c in copies:
        c.wait()

    iota = lax.iota(jnp.int32, L)

    def group_body(g, _):
        for r in range(L):
            row = g * L + r
            t = theta_v[row, pl.ds(0, L)]
            b = beta_v[row, pl.ds(0, L)]
            for c in range(1, NUM_DIM // L):
                t = t + theta_v[row, pl.ds(c * L, L)]
                b = b + beta_v[row, pl.ds(c * L, L)]
            plsc.store_scatter(tpad_v, [iota * 17 + r], t - b)
        acc = tpad_v[pl.ds(0, L)]
        for l in range(1, L):
            acc = acc + tpad_v[pl.ds(l * 17, L)]
        pred = 1.0 / (1.0 + jnp.exp(-acc))
        out_v[pl.ds(g * L, L)] = pred
        return 0

    lax.fori_loop(0, GROUPS, group_body, 0)

    pltpu.sync_copy(out_v, out_hbm.at[pl.ds(wid * B_PER_W, B_PER_W)])


@jax.jit
def _irt_sc(sid2d, qid2d, theta_weight, beta_weight):
    kern = functools.partial(
        pl.kernel,
        mesh=plsc.VectorSubcoreMesh(core_axis_name="c", subcore_axis_name="s"),
        out_type=jax.ShapeDtypeStruct((BATCH,), jnp.float32),
        compiler_params=pltpu.CompilerParams(
            needs_layout_passes=False, use_tc_tiling_on_sc=False),
        scratch_types=[
            pltpu.VMEM((N_CHUNKS, CHUNK), jnp.int32),
            pltpu.VMEM((N_CHUNKS, CHUNK), jnp.int32),
            pltpu.VMEM((B_PER_W, NUM_DIM), jnp.float32),
            pltpu.VMEM((B_PER_W, NUM_DIM), jnp.float32),
            pltpu.VMEM((L * 17,), jnp.float32),
            pltpu.VMEM((B_PER_W,), jnp.float32),
            pltpu.SemaphoreType.DMA,
        ],
    )(_body)
    return kern(sid2d, qid2d, theta_weight, beta_weight)


def kernel(student_ids, question_ids, theta_weight, beta_weight):
    sid2d = student_ids.astype(jnp.int32).reshape(NW * N_CHUNKS, CHUNK)
    qid2d = question_ids.astype(jnp.int32).reshape(NW * N_CHUNKS, CHUNK)
    out = _irt_sc(sid2d, qid2d, theta_weight, beta_weight)
    return out.reshape(BATCH, 1)


# speedup vs baseline: 1.9690x; 1.9690x over previous
import functools

import jax
import jax.numpy as jnp
from jax import lax
from jax.experimental import pallas as pl
from jax.experimental.pallas import tpu as pltpu
from jax.experimental.pallas import tpu_sc as plsc

NUM_STUDENTS = 1000000
NUM_QUESTIONS = 100000
NUM_DIM = 64
BATCH = 16384

NC = 2
NS = 16
L = 16
NW = NC * NS
B_PER_W = BATCH // NW
CHUNK = 128
N_CHUNKS = B_PER_W // CHUNK

ROWSUM_BW = 2048


def _rowsum_body(xt_ref, o_ref):
    o_ref[...] = jnp.sum(xt_ref[...], axis=0)


def _rowsum(xt):
    n = xt.shape[1]
    grid = (n + ROWSUM_BW - 1) // ROWSUM_BW
    return pl.pallas_call(
        _rowsum_body,
        grid=(grid,),
        in_specs=[pl.BlockSpec((NUM_DIM, ROWSUM_BW), lambda i: (0, i))],
        out_specs=pl.BlockSpec((ROWSUM_BW,), lambda i: (i,)),
        out_shape=jax.ShapeDtypeStruct((n,), jnp.float32),
        compiler_params=pltpu.CompilerParams(
            dimension_semantics=("arbitrary",)),
    )(xt)


def _gather_body(sid_hbm, qid_hbm, ts_hbm, bs_hbm, out_hbm,
                 sid_v, qid_v, ts_v, bs_v, out_v, sem):
    wid = lax.axis_index("s") * NC + lax.axis_index("c")

    pltpu.sync_copy(sid_hbm.at[pl.ds(wid * N_CHUNKS, N_CHUNKS)], sid_v)
    pltpu.sync_copy(qid_hbm.at[pl.ds(wid * N_CHUNKS, N_CHUNKS)], qid_v)

    copies = []
    for j in range(N_CHUNKS):
        copies.append(pltpu.async_copy(ts_hbm.at[sid_v.at[j]], ts_v.at[j], sem))
        copies.append(pltpu.async_copy(bs_hbm.at[qid_v.at[j]], bs_v.at[j], sem))
    for c in copies:
        c.wait()

    for j in range(N_CHUNKS):
        for c in range(CHUNK // L):
            diff = ts_v[j, pl.ds(c * L, L)] - bs_v[j, pl.ds(c * L, L)]
            pred = 1.0 / (1.0 + jnp.exp(-diff))
            out_v[pl.ds((j * (CHUNK // L) + c) * L, L)] = pred

    pltpu.sync_copy(out_v, out_hbm.at[pl.ds(wid * B_PER_W, B_PER_W)])


def _gather_sigmoid(sid2d, qid2d, ts, bs):
    kern = functools.partial(
        pl.kernel,
        mesh=plsc.VectorSubcoreMesh(core_axis_name="c", subcore_axis_name="s"),
        out_type=jax.ShapeDtypeStruct((BATCH,), jnp.float32),
        compiler_params=pltpu.CompilerParams(
            needs_layout_passes=False, use_tc_tiling_on_sc=False),
        scratch_types=[
            pltpu.VMEM((N_CHUNKS, CHUNK), jnp.int32),
            pltpu.VMEM((N_CHUNKS, CHUNK), jnp.int32),
            pltpu.VMEM((N_CHUNKS, CHUNK), jnp.float32),
            pltpu.VMEM((N_CHUNKS, CHUNK), jnp.float32),
            pltpu.VMEM((B_PER_W,), jnp.float32),
            pltpu.SemaphoreType.DMA,
        ],
    )(_gather_body)
    return kern(sid2d, qid2d, ts, bs)


@jax.jit
def _irt(student_ids, question_ids, theta_weight, beta_weight):
    ts = _rowsum(theta_weight.T)
    bs = _rowsum(beta_weight.T)
    sid2d = student_ids.astype(jnp.int32).reshape(NW * N_CHUNKS, CHUNK)
    qid2d = question_ids.astype(jnp.int32).reshape(NW * N_CHUNKS, CHUNK)
    return _gather_sigmoid(sid2d, qid2d, ts, bs)


def kernel(student_ids, question_ids, theta_weight, beta_weight):
    out = _irt(student_ids, question_ids, theta_weight, beta_weight)
    return out.reshape(BATCH, 1)


# speedup vs baseline: 3.2746x; 1.6631x over previous
import functools

import jax
import jax.numpy as jnp
from jax import lax
from jax.experimental import pallas as pl
from jax.experimental.pallas import tpu as pltpu
from jax.experimental.pallas import tpu_sc as plsc

NUM_STUDENTS = 1000000
NUM_QUESTIONS = 100000
NUM_DIM = 64
BATCH = 16384

NC = 2
NS = 16
L = 16
NW = NC * NS
B_PER_W = BATCH // NW
CHUNK = 128
N_CHUNKS = B_PER_W // CHUNK

ROWSUM_BW = 8192


def _rowsum_body(xt_ref, o_ref):
    ones = jnp.ones((1, NUM_DIM), jnp.float32)
    o_ref[...] = jnp.dot(ones, xt_ref[...],
                         preferred_element_type=jnp.float32)


def _rowsum(xt):
    n = xt.shape[1]
    grid = (n + ROWSUM_BW - 1) // ROWSUM_BW
    return pl.pallas_call(
        _rowsum_body,
        grid=(grid,),
        in_specs=[pl.BlockSpec((NUM_DIM, ROWSUM_BW), lambda i: (0, i))],
        out_specs=pl.BlockSpec((1, ROWSUM_BW), lambda i: (0, i)),
        out_shape=jax.ShapeDtypeStruct((1, n), jnp.float32),
        compiler_params=pltpu.CompilerParams(
            dimension_semantics=("arbitrary",)),
    )(xt)


def _gather_body(sid_hbm, qid_hbm, ts_hbm, bs_hbm, out_hbm,
                 sid_v, qid_v, ts_v, bs_v, out_v, sem):
    wid = lax.axis_index("s") * NC + lax.axis_index("c")

    pltpu.sync_copy(sid_hbm.at[pl.ds(wid * N_CHUNKS, N_CHUNKS)], sid_v)
    pltpu.sync_copy(qid_hbm.at[pl.ds(wid * N_CHUNKS, N_CHUNKS)], qid_v)

    copies = []
    for j in range(N_CHUNKS):
        copies.append(pltpu.async_copy(ts_hbm.at[sid_v.at[j]], ts_v.at[j], sem))
        copies.append(pltpu.async_copy(bs_hbm.at[qid_v.at[j]], bs_v.at[j], sem))
    for c in copies:
        c.wait()

    for j in range(N_CHUNKS):
        for c in range(CHUNK // L):
            diff = ts_v[j, pl.ds(c * L, L)] - bs_v[j, pl.ds(c * L, L)]
            pred = 1.0 / (1.0 + jnp.exp(-diff))
            out_v[pl.ds((j * (CHUNK // L) + c) * L, L)] = pred

    pltpu.sync_copy(out_v, out_hbm.at[pl.ds(wid * B_PER_W, B_PER_W)])


def _gather_sigmoid(sid2d, qid2d, ts, bs):
    kern = functools.partial(
        pl.kernel,
        mesh=plsc.VectorSubcoreMesh(core_axis_name="c", subcore_axis_name="s"),
        out_type=jax.ShapeDtypeStruct((BATCH,), jnp.float32),
        compiler_params=pltpu.CompilerParams(
            needs_layout_passes=False, use_tc_tiling_on_sc=False),
        scratch_types=[
            pltpu.VMEM((N_CHUNKS, CHUNK), jnp.int32),
            pltpu.VMEM((N_CHUNKS, CHUNK), jnp.int32),
            pltpu.VMEM((N_CHUNKS, CHUNK), jnp.float32),
            pltpu.VMEM((N_CHUNKS, CHUNK), jnp.float32),
            pltpu.VMEM((B_PER_W,), jnp.float32),
            pltpu.SemaphoreType.DMA,
        ],
    )(_gather_body)
    return kern(sid2d, qid2d, ts, bs)


@jax.jit
def _irt(student_ids, question_ids, theta_weight, beta_weight):
    ts = _rowsum(theta_weight.T).reshape(NUM_STUDENTS)
    bs = _rowsum(beta_weight.T).reshape(NUM_QUESTIONS)
    sid2d = student_ids.astype(jnp.int32).reshape(NW * N_CHUNKS, CHUNK)
    qid2d = question_ids.astype(jnp.int32).reshape(NW * N_CHUNKS, CHUNK)
    return _gather_sigmoid(sid2d, qid2d, ts, bs)


def kernel(student_ids, question_ids, theta_weight, beta_weight):
    out = _irt(student_ids, question_ids, theta_weight, beta_weight)
    return out.reshape(BATCH, 1)


# speedup vs baseline: 4.0445x; 1.2351x over previous
import functools

import jax
import jax.numpy as jnp
from jax import lax
from jax.experimental import pallas as pl
from jax.experimental.pallas import tpu as pltpu
from jax.experimental.pallas import tpu_sc as plsc

NUM_STUDENTS = 1000000
NUM_QUESTIONS = 100000
NUM_DIM = 64
BATCH = 16384

NC = 2
NS = 16
L = 16
NW = NC * NS
B_PER_W = BATCH // NW
CHUNK = 128
N_CHUNKS = B_PER_W // CHUNK

ROWSUM_BW = 16384


def _rowsum_body(xt_ref, o_ref):
    ones = jnp.ones((1, NUM_DIM), jnp.float32)
    o_ref[...] = jnp.dot(ones, xt_ref[...],
                         preferred_element_type=jnp.float32)


def _rowsum(xt):
    n = xt.shape[1]
    grid = (n + ROWSUM_BW - 1) // ROWSUM_BW
    return pl.pallas_call(
        _rowsum_body,
        grid=(grid,),
        in_specs=[pl.BlockSpec((NUM_DIM, ROWSUM_BW), lambda i: (0, i))],
        out_specs=pl.BlockSpec((1, ROWSUM_BW), lambda i: (0, i)),
        out_shape=jax.ShapeDtypeStruct((1, n), jnp.float32),
        compiler_params=pltpu.CompilerParams(
            dimension_semantics=("arbitrary",)),
    )(xt)


def _gather_body(sid_hbm, qid_hbm, ts_hbm, bs_hbm, out_hbm,
                 sid_v, qid_v, ts_v, bs_v, out_v, sem):
    wid = lax.axis_index("s") * NC + lax.axis_index("c")

    pltpu.sync_copy(sid_hbm.at[pl.ds(wid * N_CHUNKS, N_CHUNKS)], sid_v)
    pltpu.sync_copy(qid_hbm.at[pl.ds(wid * N_CHUNKS, N_CHUNKS)], qid_v)

    copies = []
    for j in range(N_CHUNKS):
        copies.append(pltpu.async_copy(ts_hbm.at[sid_v.at[j]], ts_v.at[j], sem))
        copies.append(pltpu.async_copy(bs_hbm.at[qid_v.at[j]], bs_v.at[j], sem))
    for c in copies:
        c.wait()

    for j in range(N_CHUNKS):
        for c in range(CHUNK // L):
            diff = ts_v[j, pl.ds(c * L, L)] - bs_v[j, pl.ds(c * L, L)]
            pred = 1.0 / (1.0 + jnp.exp(-diff))
            out_v[pl.ds((j * (CHUNK // L) + c) * L, L)] = pred

    pltpu.sync_copy(out_v, out_hbm.at[pl.ds(wid * B_PER_W, B_PER_W)])


def _gather_sigmoid(sid2d, qid2d, ts, bs):
    kern = functools.partial(
        pl.kernel,
        mesh=plsc.VectorSubcoreMesh(core_axis_name="c", subcore_axis_name="s"),
        out_type=jax.ShapeDtypeStruct((BATCH,), jnp.float32),
        compiler_params=pltpu.CompilerParams(
            needs_layout_passes=False, use_tc_tiling_on_sc=False),
        scratch_types=[
            pltpu.VMEM((N_CHUNKS, CHUNK), jnp.int32),
            pltpu.VMEM((N_CHUNKS, CHUNK), jnp.int32),
            pltpu.VMEM((N_CHUNKS, CHUNK), jnp.float32),
            pltpu.VMEM((N_CHUNKS, CHUNK), jnp.float32),
            pltpu.VMEM((B_PER_W,), jnp.float32),
            pltpu.SemaphoreType.DMA,
        ],
    )(_gather_body)
    return kern(sid2d, qid2d, ts, bs)


@jax.jit
def _irt(student_ids, question_ids, theta_weight, beta_weight):
    ts = _rowsum(theta_weight.T).reshape(NUM_STUDENTS)
    bs = _rowsum(beta_weight.T).reshape(NUM_QUESTIONS)
    sid2d = student_ids.astype(jnp.int32).reshape(NW * N_CHUNKS, CHUNK)
    qid2d = question_ids.astype(jnp.int32).reshape(NW * N_CHUNKS, CHUNK)
    return _gather_sigmoid(sid2d, qid2d, ts, bs)


def kernel(student_ids, question_ids, theta_weight, beta_weight):
    out = _irt(student_ids, question_ids, theta_weight, beta_weight)
    return out.reshape(BATCH, 1)


# speedup vs baseline: 4.3131x; 1.0664x over previous
import functools

import jax
import jax.numpy as jnp
from jax import lax
from jax.experimental import pallas as pl
from jax.experimental.pallas import tpu as pltpu
from jax.experimental.pallas import tpu_sc as plsc

NUM_STUDENTS = 1000000
NUM_QUESTIONS = 100000
NUM_DIM = 64
BATCH = 16384

NC = 2
NS = 16
L = 16
NW = NC * NS
B_PER_W = BATCH // NW
CHUNK = 128
N_CHUNKS = B_PER_W // CHUNK

ROWSUM_BW = 32768


def _rowsum_body(xt_ref, o_ref):
    ones = jnp.ones((1, NUM_DIM), jnp.float32)
    o_ref[...] = jnp.dot(ones, xt_ref[...],
                         preferred_element_type=jnp.float32)


def _rowsum(xt):
    n = xt.shape[1]
    grid = (n + ROWSUM_BW - 1) // ROWSUM_BW
    return pl.pallas_call(
        _rowsum_body,
        grid=(grid,),
        in_specs=[pl.BlockSpec((NUM_DIM, ROWSUM_BW), lambda i: (0, i))],
        out_specs=pl.BlockSpec((1, ROWSUM_BW), lambda i: (0, i)),
        out_shape=jax.ShapeDtypeStruct((1, n), jnp.float32),
        compiler_params=pltpu.CompilerParams(
            dimension_semantics=("arbitrary",)),
    )(xt)


def _gather_body(sid_hbm, qid_hbm, ts_hbm, bs_hbm, out_hbm,
                 sid_v, qid_v, ts_v, bs_v, out_v, sem):
    wid = lax.axis_index("s") * NC + lax.axis_index("c")

    pltpu.sync_copy(sid_hbm.at[pl.ds(wid * N_CHUNKS, N_CHUNKS)], sid_v)
    pltpu.sync_copy(qid_hbm.at[pl.ds(wid * N_CHUNKS, N_CHUNKS)], qid_v)

    copies = []
    for j in range(N_CHUNKS):
        copies.append(pltpu.async_copy(ts_hbm.at[sid_v.at[j]], ts_v.at[j], sem))
        copies.append(pltpu.async_copy(bs_hbm.at[qid_v.at[j]], bs_v.at[j], sem))
    for c in copies:
        c.wait()

    for j in range(N_CHUNKS):
        for c in range(CHUNK // L):
            diff = ts_v[j, pl.ds(c * L, L)] - bs_v[j, pl.ds(c * L, L)]
            pred = 1.0 / (1.0 + jnp.exp(-diff))
            out_v[pl.ds((j * (CHUNK // L) + c) * L, L)] = pred

    pltpu.sync_copy(out_v, out_hbm.at[pl.ds(wid * B_PER_W, B_PER_W)])


def _gather_sigmoid(sid2d, qid2d, ts, bs):
    kern = functools.partial(
        pl.kernel,
        mesh=plsc.VectorSubcoreMesh(core_axis_name="c", subcore_axis_name="s"),
        out_type=jax.ShapeDtypeStruct((BATCH,), jnp.float32),
        compiler_params=pltpu.CompilerParams(
            needs_layout_passes=False, use_tc_tiling_on_sc=False),
        scratch_types=[
            pltpu.VMEM((N_CHUNKS, CHUNK), jnp.int32),
            pltpu.VMEM((N_CHUNKS, CHUNK), jnp.int32),
            pltpu.VMEM((N_CHUNKS, CHUNK), jnp.float32),
            pltpu.VMEM((N_CHUNKS, CHUNK), jnp.float32),
            pltpu.VMEM((B_PER_W,), jnp.float32),
            pltpu.SemaphoreType.DMA,
        ],
    )(_gather_body)
    return kern(sid2d, qid2d, ts, bs)


@jax.jit
def _irt(student_ids, question_ids, theta_weight, beta_weight):
    ts = _rowsum(theta_weight.T).reshape(NUM_STUDENTS)
    bs = _rowsum(beta_weight.T).reshape(NUM_QUESTIONS)
    sid2d = student_ids.astype(jnp.int32).reshape(NW * N_CHUNKS, CHUNK)
    qid2d = question_ids.astype(jnp.int32).reshape(NW * N_CHUNKS, CHUNK)
    return _gather_sigmoid(sid2d, qid2d, ts, bs)


def kernel(student_ids, question_ids, theta_weight, beta_weight):
    out = _irt(student_ids, question_ids, theta_weight, beta_weight)
    return out.reshape(BATCH, 1)
